# baseline (speedup 1.0000x reference)
import functools

import jax
import jax.numpy as jnp
from jax import lax
from jax.experimental import pallas as pl
from jax.experimental.pallas import tpu as pltpu
from jax.experimental.pallas import tpu_sc as plsc

DIM = 1024
BINS = 1024
TOK_TILE = 512
NB = 1


def _first_step():
    return jnp.logical_and(pl.program_id(0) == 0, pl.program_id(1) == 0)


def _codes_from(r, cb_ref, cn_ref):
    @pl.when(_first_step())
    def _():
        cb = cb_ref[0]
        cn_ref[...] = jnp.sum(cb * cb, axis=1, keepdims=True)

    s = lax.dot_general(cb_ref[0], r, (((1,), (0,)), ((), ())),
                        preferred_element_type=jnp.float32)
    rn = jnp.sum(r * r, axis=0, keepdims=True)
    d = (rn - 2.0 * s) + cn_ref[...]
    m = jnp.min(d, axis=0, keepdims=True)
    iota = lax.broadcasted_iota(jnp.int32, d.shape, 0)
    cand = jnp.where(d == m, iota, BINS)
    return jnp.min(cand, axis=0, keepdims=True)


def _stage_a_body(x_ref, cb_ref, codes_ref, cn_ref):
    codes_ref[...] = _codes_from(x_ref[0], cb_ref, cn_ref).reshape(1, 1, TOK_TILE)


def _stage_b_body(x_ref, q0_ref, cb_ref, codes_ref, cn_ref):
    r = x_ref[0] - q0_ref[...].T
    codes_ref[...] = _codes_from(r, cb_ref, cn_ref).reshape(1, 1, TOK_TILE)


def _stage_c_body(x_ref, q0_ref, q1_ref, quant_ref, res2_ref, l0_ref, l1_ref):
    xt = x_ref[0]
    r1 = xt - q0_ref[...].T
    r2 = r1 - q1_ref[...].T
    res2_ref[0] = r2
    quant_ref[0] = xt - r2

    @pl.when(_first_step())
    def _():
        l0_ref[...] = jnp.zeros_like(l0_ref)
        l1_ref[...] = jnp.zeros_like(l1_ref)

    l0_ref[...] += jnp.sum(r1 * r1).reshape(1, 1)
    l1_ref[...] += jnp.sum(r2 * r2).reshape(1, 1)


def _x_spec(boff):
    return pl.BlockSpec((1, DIM, TOK_TILE), lambda b, t: (b + boff, 0, t))


def _q_spec(nt):
    return pl.BlockSpec((TOK_TILE, DIM), lambda b, t: (b * nt + t, 0))


def _cb_spec(q):
    return pl.BlockSpec((1, BINS, DIM), lambda b, t: (q, 0, 0))


def _codes_spec(nt):
    return pl.BlockSpec((1, 1, TOK_TILE), lambda b, t: (b * nt + t, 0, 0))


def _loss_spec():
    return pl.BlockSpec((1, 1), lambda b, t: (0, 0))


def _stage_a(x, cbs, boff):
    tt = x.shape[2]
    nt = tt // TOK_TILE
    return pl.pallas_call(
        _stage_a_body,
        grid=(NB, nt),
        in_specs=[_x_spec(boff), _cb_spec(0)],
        out_specs=_codes_spec(nt),
        out_shape=jax.ShapeDtypeStruct((NB * nt, 1, TOK_TILE), jnp.int32),
        scratch_shapes=[pltpu.VMEM((BINS, 1), jnp.float32)],
    )(x, cbs)


def _stage_b(x, q0, cbs, boff):
    tt = x.shape[2]
    nt = tt // TOK_TILE
    return pl.pallas_call(
        _stage_b_body,
        grid=(NB, nt),
        in_specs=[_x_spec(boff), _q_spec(nt), _cb_spec(1)],
        out_specs=_codes_spec(nt),
        out_shape=jax.ShapeDtypeStruct((NB * nt, 1, TOK_TILE), jnp.int32),
        scratch_shapes=[pltpu.VMEM((BINS, 1), jnp.float32)],
    )(x, q0, cbs)


C_TILE = 1024


def _stage_c(x, q0, q1, boff, prev=None):
    bb, d, tt = x.shape
    nt = tt // C_TILE
    xs = pl.BlockSpec((1, DIM, C_TILE), lambda b, t: (b + boff, 0, t))
    qs = pl.BlockSpec((C_TILE, DIM), lambda b, t: (b * nt + t, 0))
    in_specs = [xs, qs, qs]
    operands = [x, q0, q1]
    aliases = {}
    if prev is not None:
        any_spec = pl.BlockSpec(memory_space=pl.ANY)
        in_specs += [any_spec, any_spec]
        operands += [prev[0], prev[1]]
        aliases = {3: 0, 4: 1}

    def body(*refs):
        if prev is not None:
            x_ref, q0_ref, q1_ref, _, _, quant_ref, res2_ref, l0_ref, l1_ref = refs
        else:
            x_ref, q0_ref, q1_ref, quant_ref, res2_ref, l0_ref, l1_ref = refs
        _stage_c_body(x_ref, q0_ref, q1_ref, quant_ref, res2_ref, l0_ref, l1_ref)

    return pl.pallas_call(
        body,
        grid=(NB, nt),
        in_specs=in_specs,
        out_specs=[xs, xs, _loss_spec(), _loss_spec()],
        out_shape=[
            jax.ShapeDtypeStruct((bb, d, tt), jnp.float32),
            jax.ShapeDtypeStruct((bb, d, tt), jnp.float32),
            jax.ShapeDtypeStruct((1, 1), jnp.float32),
            jax.ShapeDtypeStruct((1, 1), jnp.float32),
        ],
        input_output_aliases=aliases,
    )(*operands)


def _sc_gather(cbs, q, idx):
    n = idx.shape[0]
    info = plsc.get_sparse_core_info()
    nw = info.num_cores * info.num_subcores
    b_per_w = n // nw
    nch = 4
    ch = b_per_w // nch

    mesh = plsc.VectorSubcoreMesh(core_axis_name="c", subcore_axis_name="s")

    @functools.partial(
        pl.kernel,
        mesh=mesh,
        out_type=jax.ShapeDtypeStruct((n, DIM), jnp.float32),
    scratch_types=[
            pltpu.VMEM((b_per_w,), jnp.int32),
            pltpu.VMEM((nch, ch, DIM), jnp.float32),
            pltpu.SemaphoreType.DMA,
            pltpu.SemaphoreType.DMA,
        ],
    )
    def k(cbs_hbm, idx_hbm, out_hbm, idx_v, rows_v, gsem, wsem):
        wid = lax.axis_index("s") * info.num_cores + lax.axis_index("c")
        base = wid * b_per_w
        table = cbs_hbm.at[q]
        pltpu.sync_copy(idx_hbm.at[pl.ds(base, b_per_w)], idx_v)
        gh = [
            pltpu.async_copy(
                table.at[idx_v.at[pl.ds(c * ch, ch)]], rows_v.at[c], gsem
            )
            for c in range(nch)
        ]
        wh = []
        for c in range(nch):
            gh[c].wait()
            wh.append(
                pltpu.async_copy(
                    rows_v.at[c], out_hbm.at[pl.ds(base + c * ch, ch)], wsem
                )
            )
        for h in wh:
            h.wait()

    return k(cbs, idx)


def kernel(x, codebooks):
    bb, d, tt = x.shape
    n = bb * tt
    nh = NB * tt

    nchunk = bb // NB
    c0 = [_stage_a(x, codebooks, i * NB) for i in range(nchunk)]
    q0 = [_sc_gather(codebooks, 0, c.reshape(nh)) for c in c0]
    c1 = [_stage_b(x, q0[i], codebooks, i * NB) for i in range(nchunk)]
    q1 = [_sc_gather(codebooks, 1, c.reshape(nh)) for c in c1]
    prev = None
    losses = []
    for i in range(nchunk):
        quant, res, l0, l1 = _stage_c(x, q0[i], q1[i], i * NB, prev=prev)
        prev = (quant, res)
        losses += [l0, l1]
    quantized, residual_out = prev

    commit_loss = sum(l[0, 0] for l in losses) / (2.0 * n * d)
    codes = jnp.stack([
        jnp.concatenate([c.reshape(NB, tt) for c in c0]),
        jnp.concatenate([c.reshape(NB, tt) for c in c1]),
    ])
    return quantized, residual_out, commit_loss, codes


# speedup vs baseline: 1.1768x; 1.1768x over previous
import functools

import jax
import jax.numpy as jnp
from jax import lax
from jax.experimental import pallas as pl
from jax.experimental.pallas import tpu as pltpu
from jax.experimental.pallas import tpu_sc as plsc

DIM = 1024
BINS = 1024
TOK_TILE = 512
NB = 2


def _first_step():
    return jnp.logical_and(pl.program_id(0) == 0, pl.program_id(1) == 0)


def _codes_from(r, cb_ref, cn_ref):
    @pl.when(_first_step())
    def _():
        cb = cb_ref[0]
        cn_ref[...] = jnp.sum(cb * cb, axis=1, keepdims=True)

    s = lax.dot_general(cb_ref[0], r, (((1,), (0,)), ((), ())),
                        preferred_element_type=jnp.float32)
    rn = jnp.sum(r * r, axis=0, keepdims=True)
    d = (rn - 2.0 * s) + cn_ref[...]
    m = jnp.min(d, axis=0, keepdims=True)
    iota = lax.broadcasted_iota(jnp.int32, d.shape, 0)
    cand = jnp.where(d == m, iota, BINS)
    return jnp.min(cand, axis=0, keepdims=True)


def _stage_a_body(x_ref, cb_ref, codes_ref, cn_ref):
    codes_ref[...] = _codes_from(x_ref[0], cb_ref, cn_ref).reshape(1, 1, TOK_TILE)


def _stage_b_body(x_ref, q0_ref, cb_ref, codes_ref, cn_ref):
    r = x_ref[0] - q0_ref[...].T
    codes_ref[...] = _codes_from(r, cb_ref, cn_ref).reshape(1, 1, TOK_TILE)


def _stage_c_body(x_ref, q0_ref, q1_ref, quant_ref, res2_ref, l0_ref, l1_ref):
    xt = x_ref[0]
    r1 = xt - q0_ref[...].T
    r2 = r1 - q1_ref[...].T
    res2_ref[0] = r2
    quant_ref[0] = xt - r2

    @pl.when(_first_step())
    def _():
        l0_ref[...] = jnp.zeros_like(l0_ref)
        l1_ref[...] = jnp.zeros_like(l1_ref)

    l0_ref[...] += jnp.sum(r1 * r1).reshape(1, 1)
    l1_ref[...] += jnp.sum(r2 * r2).reshape(1, 1)


def _x_spec(boff):
    return pl.BlockSpec((1, DIM, TOK_TILE), lambda b, t: (b + boff, 0, t))


def _q_spec(nt):
    return pl.BlockSpec((TOK_TILE, DIM), lambda b, t: (b * nt + t, 0))


def _cb_spec(q):
    return pl.BlockSpec((1, BINS, DIM), lambda b, t: (q, 0, 0))


def _codes_spec(nt):
    return pl.BlockSpec((1, 1, TOK_TILE), lambda b, t: (b * nt + t, 0, 0))


def _loss_spec():
    return pl.BlockSpec((1, 1), lambda b, t: (0, 0))


def _stage_a(x, cbs, boff):
    tt = x.shape[2]
    nt = tt // TOK_TILE
    return pl.pallas_call(
        _stage_a_body,
        grid=(NB, nt),
        in_specs=[_x_spec(boff), _cb_spec(0)],
        out_specs=_codes_spec(nt),
        out_shape=jax.ShapeDtypeStruct((NB * nt, 1, TOK_TILE), jnp.int32),
        scratch_shapes=[pltpu.VMEM((BINS, 1), jnp.float32)],
    )(x, cbs)


def _stage_b(x, q0, cbs, boff):
    tt = x.shape[2]
    nt = tt // TOK_TILE
    return pl.pallas_call(
        _stage_b_body,
        grid=(NB, nt),
        in_specs=[_x_spec(boff), _q_spec(nt), _cb_spec(1)],
        out_specs=_codes_spec(nt),
        out_shape=jax.ShapeDtypeStruct((NB * nt, 1, TOK_TILE), jnp.int32),
        scratch_shapes=[pltpu.VMEM((BINS, 1), jnp.float32)],
    )(x, q0, cbs)


C_TILE = 1024


def _stage_c(x, q0, q1, boff, prev=None):
    bb, d, tt = x.shape
    nt = tt // C_TILE
    xs = pl.BlockSpec((1, DIM, C_TILE), lambda b, t: (b + boff, 0, t))
    qs = pl.BlockSpec((C_TILE, DIM), lambda b, t: (b * nt + t, 0))
    in_specs = [xs, qs, qs]
    operands = [x, q0, q1]
    aliases = {}
    if prev is not None:
        any_spec = pl.BlockSpec(memory_space=pl.ANY)
        in_specs += [any_spec, any_spec]
        operands += [prev[0], prev[1]]
        aliases = {3: 0, 4: 1}

    def body(*refs):
        if prev is not None:
            x_ref, q0_ref, q1_ref, _, _, quant_ref, res2_ref, l0_ref, l1_ref = refs
        else:
            x_ref, q0_ref, q1_ref, quant_ref, res2_ref, l0_ref, l1_ref = refs
        _stage_c_body(x_ref, q0_ref, q1_ref, quant_ref, res2_ref, l0_ref, l1_ref)

    return pl.pallas_call(
        body,
        grid=(NB, nt),
        in_specs=in_specs,
        out_specs=[xs, xs, _loss_spec(), _loss_spec()],
        out_shape=[
            jax.ShapeDtypeStruct((bb, d, tt), jnp.float32),
            jax.ShapeDtypeStruct((bb, d, tt), jnp.float32),
            jax.ShapeDtypeStruct((1, 1), jnp.float32),
            jax.ShapeDtypeStruct((1, 1), jnp.float32),
        ],
        input_output_aliases=aliases,
    )(*operands)


def _sc_gather(cbs, q, idx):
    n = idx.shape[0]
    info = plsc.get_sparse_core_info()
    nw = info.num_cores * info.num_subcores
    b_per_w = n // nw
    nch = 4
    ch = b_per_w // nch

    mesh = plsc.VectorSubcoreMesh(core_axis_name="c", subcore_axis_name="s")

    @functools.partial(
        pl.kernel,
        mesh=mesh,
        out_type=jax.ShapeDtypeStruct((n, DIM), jnp.float32),
    scratch_types=[
            pltpu.VMEM((b_per_w,), jnp.int32),
            pltpu.VMEM((nch, ch, DIM), jnp.float32),
            pltpu.SemaphoreType.DMA,
            pltpu.SemaphoreType.DMA,
        ],
    )
    def k(cbs_hbm, idx_hbm, out_hbm, idx_v, rows_v, gsem, wsem):
        wid = lax.axis_index("s") * info.num_cores + lax.axis_index("c")
        base = wid * b_per_w
        table = cbs_hbm.at[q]
        pltpu.sync_copy(idx_hbm.at[pl.ds(base, b_per_w)], idx_v)
        gh = [
            pltpu.async_copy(
                table.at[idx_v.at[pl.ds(c * ch, ch)]], rows_v.at[c], gsem
            )
            for c in range(nch)
        ]
        wh = []
        for c in range(nch):
            gh[c].wait()
            wh.append(
                pltpu.async_copy(
                    rows_v.at[c], out_hbm.at[pl.ds(base + c * ch, ch)], wsem
                )
            )
        for h in wh:
            h.wait()

    return k(cbs, idx)


def kernel(x, codebooks):
    bb, d, tt = x.shape
    n = bb * tt
    nh = NB * tt

    c0a = _stage_a(x, codebooks, 0)
    c0b = _stage_a(x, codebooks, NB)
    q0a = _sc_gather(codebooks, 0, c0a.reshape(nh))
    q0b = _sc_gather(codebooks, 0, c0b.reshape(nh))
    c1a = _stage_b(x, q0a, codebooks, 0)
    c1b = _stage_b(x, q0b, codebooks, NB)
    q1a = _sc_gather(codebooks, 1, c1a.reshape(nh))
    q1b = _sc_gather(codebooks, 1, c1b.reshape(nh))
    quant_a, res_a, l0a, l1a = _stage_c(x, q0a, q1a, 0)
    quantized, residual_out, l0b, l1b = _stage_c(
        x, q0b, q1b, NB, prev=(quant_a, res_a))

    commit_loss = (l0a[0, 0] + l0b[0, 0] + l1a[0, 0] + l1b[0, 0]) / (2.0 * n * d)
    codes = jnp.stack([
        jnp.concatenate([c0a.reshape(NB, tt), c0b.reshape(NB, tt)]),
        jnp.concatenate([c1a.reshape(NB, tt), c1b.reshape(NB, tt)]),
    ])
    return quantized, residual_out, commit_loss, codes


# speedup vs baseline: 1.2680x; 1.0775x over previous
import functools

import jax
import jax.numpy as jnp
from jax import lax
from jax.experimental import pallas as pl
from jax.experimental.pallas import tpu as pltpu
from jax.experimental.pallas import tpu_sc as plsc

DIM = 1024
BINS = 1024
TOK_TILE = 1024
NB = 2


def _first_step():
    return jnp.logical_and(pl.program_id(0) == 0, pl.program_id(1) == 0)


def _codes_from(r, cb_ref, cn_ref):
    @pl.when(_first_step())
    def _():
        cb = cb_ref[0]
        cn_ref[...] = jnp.sum(cb * cb, axis=1, keepdims=True)

    s = lax.dot_general(cb_ref[0], r, (((1,), (0,)), ((), ())),
                        preferred_element_type=jnp.float32)
    rn = jnp.sum(r * r, axis=0, keepdims=True)
    d = (rn - 2.0 * s) + cn_ref[...]
    m = jnp.min(d, axis=0, keepdims=True)
    iota = lax.broadcasted_iota(jnp.int32, d.shape, 0)
    cand = jnp.where(d == m, iota, BINS)
    return jnp.min(cand, axis=0, keepdims=True)


def _stage_a_body(x_ref, cb_ref, codes_ref, cn_ref):
    codes_ref[...] = _codes_from(x_ref[0], cb_ref, cn_ref).reshape(1, 1, TOK_TILE)


def _stage_b_body(x_ref, q0_ref, cb_ref, codes_ref, cn_ref):
    r = x_ref[0] - q0_ref[...].T
    codes_ref[...] = _codes_from(r, cb_ref, cn_ref).reshape(1, 1, TOK_TILE)


def _stage_c_body(x_ref, q0_ref, q1_ref, quant_ref, res2_ref, l0_ref, l1_ref):
    xt = x_ref[0]
    r1 = xt - q0_ref[...].T
    r2 = r1 - q1_ref[...].T
    res2_ref[0] = r2
    quant_ref[0] = xt - r2

    @pl.when(_first_step())
    def _():
        l0_ref[...] = jnp.zeros_like(l0_ref)
        l1_ref[...] = jnp.zeros_like(l1_ref)

    l0_ref[...] += jnp.sum(r1 * r1).reshape(1, 1)
    l1_ref[...] += jnp.sum(r2 * r2).reshape(1, 1)


def _x_spec(boff):
    return pl.BlockSpec((1, DIM, TOK_TILE), lambda b, t: (b + boff, 0, t))


def _q_spec(nt):
    return pl.BlockSpec((TOK_TILE, DIM), lambda b, t: (b * nt + t, 0))


def _cb_spec(q):
    return pl.BlockSpec((1, BINS, DIM), lambda b, t: (q, 0, 0))


def _codes_spec(nt):
    return pl.BlockSpec((1, 1, TOK_TILE), lambda b, t: (b * nt + t, 0, 0))


def _loss_spec():
    return pl.BlockSpec((1, 1), lambda b, t: (0, 0))


def _stage_a(x, cbs, boff):
    tt = x.shape[2]
    nt = tt // TOK_TILE
    return pl.pallas_call(
        _stage_a_body,
        grid=(NB, nt),
        in_specs=[_x_spec(boff), _cb_spec(0)],
        out_specs=_codes_spec(nt),
        out_shape=jax.ShapeDtypeStruct((NB * nt, 1, TOK_TILE), jnp.int32),
        scratch_shapes=[pltpu.VMEM((BINS, 1), jnp.float32)],
    )(x, cbs)


def _stage_b(x, q0, cbs, boff):
    tt = x.shape[2]
    nt = tt // TOK_TILE
    return pl.pallas_call(
        _stage_b_body,
        grid=(NB, nt),
        in_specs=[_x_spec(boff), _q_spec(nt), _cb_spec(1)],
        out_specs=_codes_spec(nt),
        out_shape=jax.ShapeDtypeStruct((NB * nt, 1, TOK_TILE), jnp.int32),
        scratch_shapes=[pltpu.VMEM((BINS, 1), jnp.float32)],
    )(x, q0, cbs)


C_TILE = 1024


def _stage_c(x, q0, q1, boff, prev=None):
    bb, d, tt = x.shape
    nt = tt // C_TILE
    xs = pl.BlockSpec((1, DIM, C_TILE), lambda b, t: (b + boff, 0, t))
    qs = pl.BlockSpec((C_TILE, DIM), lambda b, t: (b * nt + t, 0))
    in_specs = [xs, qs, qs]
    operands = [x, q0, q1]
    aliases = {}
    if prev is not None:
        any_spec = pl.BlockSpec(memory_space=pl.ANY)
        in_specs += [any_spec, any_spec]
        operands += [prev[0], prev[1]]
        aliases = {3: 0, 4: 1}

    def body(*refs):
        if prev is not None:
            x_ref, q0_ref, q1_ref, _, _, quant_ref, res2_ref, l0_ref, l1_ref = refs
        else:
            x_ref, q0_ref, q1_ref, quant_ref, res2_ref, l0_ref, l1_ref = refs
        _stage_c_body(x_ref, q0_ref, q1_ref, quant_ref, res2_ref, l0_ref, l1_ref)

    return pl.pallas_call(
        body,
        grid=(NB, nt),
        in_specs=in_specs,
        out_specs=[xs, xs, _loss_spec(), _loss_spec()],
        out_shape=[
            jax.ShapeDtypeStruct((bb, d, tt), jnp.float32),
            jax.ShapeDtypeStruct((bb, d, tt), jnp.float32),
            jax.ShapeDtypeStruct((1, 1), jnp.float32),
            jax.ShapeDtypeStruct((1, 1), jnp.float32),
        ],
        input_output_aliases=aliases,
    )(*operands)


def _sc_gather(cbs, q, idx):
    n = idx.shape[0]
    info = plsc.get_sparse_core_info()
    nw = info.num_cores * info.num_subcores
    b_per_w = n // nw
    nch = 4
    ch = b_per_w // nch

    mesh = plsc.VectorSubcoreMesh(core_axis_name="c", subcore_axis_name="s")

    @functools.partial(
        pl.kernel,
        mesh=mesh,
        out_type=jax.ShapeDtypeStruct((n, DIM), jnp.float32),
    scratch_types=[
            pltpu.VMEM((b_per_w,), jnp.int32),
            pltpu.VMEM((nch, ch, DIM), jnp.float32),
            pltpu.SemaphoreType.DMA,
            pltpu.SemaphoreType.DMA,
        ],
    )
    def k(cbs_hbm, idx_hbm, out_hbm, idx_v, rows_v, gsem, wsem):
        wid = lax.axis_index("s") * info.num_cores + lax.axis_index("c")
        base = wid * b_per_w
        table = cbs_hbm.at[q]
        pltpu.sync_copy(idx_hbm.at[pl.ds(base, b_per_w)], idx_v)
        gh = [
            pltpu.async_copy(
                table.at[idx_v.at[pl.ds(c * ch, ch)]], rows_v.at[c], gsem
            )
            for c in range(nch)
        ]
        wh = []
        for c in range(nch):
            gh[c].wait()
            wh.append(
                pltpu.async_copy(
                    rows_v.at[c], out_hbm.at[pl.ds(base + c * ch, ch)], wsem
                )
            )
        for h in wh:
            h.wait()

    return k(cbs, idx)


def kernel(x, codebooks):
    bb, d, tt = x.shape
    n = bb * tt
    nh = NB * tt

    c0a = _stage_a(x, codebooks, 0)
    c0b = _stage_a(x, codebooks, NB)
    q0a = _sc_gather(codebooks, 0, c0a.reshape(nh))
    q0b = _sc_gather(codebooks, 0, c0b.reshape(nh))
    c1a = _stage_b(x, q0a, codebooks, 0)
    c1b = _stage_b(x, q0b, codebooks, NB)
    q1a = _sc_gather(codebooks, 1, c1a.reshape(nh))
    q1b = _sc_gather(codebooks, 1, c1b.reshape(nh))
    quant_a, res_a, l0a, l1a = _stage_c(x, q0a, q1a, 0)
    quantized, residual_out, l0b, l1b = _stage_c(
        x, q0b, q1b, NB, prev=(quant_a, res_a))

    commit_loss = (l0a[0, 0] + l0b[0, 0] + l1a[0, 0] + l1b[0, 0]) / (2.0 * n * d)
    codes = jnp.stack([
        jnp.concatenate([c0a.reshape(NB, tt), c0b.reshape(NB, tt)]),
        jnp.concatenate([c1a.reshape(NB, tt), c1b.reshape(NB, tt)]),
    ])
    return quantized, residual_out, commit_loss, codes
